# baseline (speedup 1.0000x reference)
import functools

import jax
import jax.numpy as jnp
from jax import lax
from jax.experimental import pallas as pl
from jax.experimental.pallas import tpu as pltpu
from jax.experimental.pallas import tpu_sc as plsc

_B, _S, _D = 4, 8192, 768
_HB = 2
_K = 1638
_KPAD = 1664
_NCH = 16
_SEG = _KPAD // _NCH
_SEGV = (_SEG + 15) // 16
_BLK = 4096
_HBLK = (_HB * _S) // _BLK
_C = _S // _NCH
_NVC = _C // 16
_NBINS = 256


def _gate_body(x_ref, w_ref, b_ref, out_ref, sp_ref, ent_ref):
    i = pl.program_id(0)
    lg = lax.dot_general(
        x_ref[...], w_ref[...], (((1,), (0,)), ((), ())),
        preferred_element_type=jnp.float32,
    ) + b_ref[0, 0]
    out_ref[...] = lg
    p = jax.nn.sigmoid(lg)
    ent = -(p * jnp.log(p + 1e-10) + (1.0 - p) * jnp.log(1.0 - p + 1e-10))
    sp = jnp.sum(p)
    en = jnp.sum(ent)

    @pl.when(i == 0)
    def _init():
        sp_ref[0, 0] = sp
        ent_ref[0, 0] = en

    @pl.when(i != 0)
    def _acc():
        sp_ref[0, 0] += sp
        ent_ref[0, 0] += en


def _make_gate(half):
    return pl.pallas_call(
        _gate_body,
        grid=(_HBLK,),
        in_specs=[
            pl.BlockSpec((_BLK, _D), lambda i: (half * _HBLK + i, 0)),
            pl.BlockSpec((_D, 1), lambda i: (0, 0)),
            pl.BlockSpec(memory_space=pltpu.SMEM),
        ],
        out_specs=[
            pl.BlockSpec((_BLK, 1), lambda i: (i, 0)),
            pl.BlockSpec(memory_space=pltpu.SMEM),
            pl.BlockSpec(memory_space=pltpu.SMEM),
        ],
        out_shape=[
            jax.ShapeDtypeStruct((_HB * _S, 1), jnp.float32),
            jax.ShapeDtypeStruct((1, 1), jnp.float32),
            jax.ShapeDtypeStruct((1, 1), jnp.float32),
        ],
        compiler_params=pltpu.CompilerParams(
            dimension_semantics=("arbitrary",),
        ),
    )


def _select_body(logits_hbm, mask_hbm, idx_hbm,
                 lrow, keys, hist, cbuf, lck, ck1, ck2, asm,
                 mrow, ibuf, ibuf2, tmp, cnt,
                 sh_hist, sh_nck, sh_ck, sh_idx, sh_cnt):
    cid = lax.axis_index("s")
    lr = lax.axis_index("c")

    lane = lax.iota(jnp.int32, 16)
    zeros16 = jnp.zeros((16,), jnp.int32)
    ones16 = jnp.ones((16,), jnp.int32)
    sign = jnp.int32(-2147483648)

    def zero_hist():
        def zbody(i, cc):
            hist[pl.ds(i * 16, 16)] = zeros16
            return cc
        lax.fori_loop(0, (16 * _NBINS) // 16, zbody, jnp.int32(0))

    def splat(x):
        return jnp.full((16,), x, jnp.int32)

    pltpu.sync_copy(logits_hbm.at[pl.ds(lr * _S + cid * _C, _C)], lrow)
    zero_hist()

    def p1body(i, cc):
        for u in range(4):
            j = i * 4 + u
            bits = lax.bitcast_convert_type(lrow[pl.ds(j * 16, 16)], jnp.int32)
            key = bits ^ (lax.shift_right_arithmetic(bits, 31) & jnp.int32(0x7FFFFFFF))
            key = key + jnp.where(key == jnp.int32(-1), jnp.int32(1), jnp.int32(0))
            keys[pl.ds(j * 16, 16)] = key
            binv = lax.shift_right_logical(key ^ sign, jnp.int32(24))
            plsc.addupdate_scatter(hist, [lane * _NBINS + binv], ones16)
        return cc

    lax.fori_loop(0, _NVC // 4, p1body, jnp.int32(0))

    def rbody(i, cc):
        tot = hist[pl.ds(i * 16, 16)]
        for j in range(1, 16):
            tot = tot + hist[pl.ds(j * _NBINS + i * 16, 16)]
        cbuf[pl.ds(i * 16, 16)] = tot
        return cc

    lax.fori_loop(0, _NBINS // 16, rbody, jnp.int32(0))
    pltpu.sync_copy(cbuf, sh_hist.at[pl.ds(cid * _NBINS, _NBINS)])
    plsc.subcore_barrier()

    def level_select_from_cbuf(n, k_rem):
        excess = n - k_rem

        def bbody(i, acc):
            cc = cbuf[pl.ds(i * 16, 16)]
            return acc + jnp.where(cc <= excess, jnp.int32(1), jnp.int32(0))

        accv = lax.fori_loop(0, _NBINS // 16, bbody, zeros16)
        bstar = jnp.sum(accv)
        cb = plsc.load_gather(cbuf, [splat(bstar)])
        g_above = n - jnp.max(cb)
        return bstar, g_above

    pltpu.sync_copy(sh_hist.at[pl.ds(0, _NCH * _NBINS)],
                    asm.at[pl.ds(0, _NCH * _NBINS)])

    def sbody0(i, csum):
        tot = asm[pl.ds(i * 16, 16)]
        for j in range(1, _NCH):
            tot = tot + asm[pl.ds(j * _NBINS + i * 16, 16)]
        cbuf[pl.ds(i * 16, 16)] = plsc.cumsum(tot) + csum
        return csum + jnp.sum(tot)

    n0 = lax.fori_loop(0, _NBINS // 16, sbody0, jnp.int32(0))
    b0, g0 = level_select_from_cbuf(n0, jnp.int32(_K))

    def c0body(i, off_vec):
        for u in range(4):
            j = i * 4 + u
            key = keys[pl.ds(j * 16, 16)]
            binv = lax.shift_right_logical(key ^ sign, jnp.int32(24))
            m = binv == b0
            mi = jnp.where(m, jnp.int32(1), jnp.int32(0))
            pos = jnp.maximum(plsc.cumsum(mi) + off_vec, jnp.int32(0))
            plsc.store_scatter(lck, [pos], key, mask=m)
            off_vec = off_vec + plsc.all_reduce_population_count(m)
        return off_vec

    offv = lax.fori_loop(0, _NVC // 4, c0body, jnp.full((16,), -1, jnp.int32))
    nck = jnp.max(offv) + jnp.int32(1)
    tmp[pl.ds(0, 16)] = splat(nck)
    pltpu.sync_copy(tmp.at[pl.ds(0, 16)], sh_nck.at[pl.ds(cid * 16, 16)])
    pltpu.sync_copy(lck, sh_ck.at[pl.ds(cid * _C, _C)])
    plsc.subcore_barrier()

    pltpu.sync_copy(sh_ck.at[pl.ds(0, _NCH * _C)], asm)
    pltpu.sync_copy(sh_nck.at[pl.ds(0, _NCH * 16)],
                    cnt.at[pl.ds(0, _NCH * 16)])
    k_rem = jnp.int32(_K) - g0

    off_vec = jnp.full((16,), -1, jnp.int32)
    for ch in range(_NCH):
        n_ch = cnt[pl.ds(ch * 16, 16)][0]
        nspl = splat(n_ch)

        def abody(i, ov, ch=ch, nspl=nspl):
            m = (lane + i * 16) < nspl
            key = asm[pl.ds(ch * _C + i * 16, 16)]
            mi = jnp.where(m, jnp.int32(1), jnp.int32(0))
            pos = jnp.maximum(plsc.cumsum(mi) + ov, jnp.int32(0))
            plsc.store_scatter(ck1, [pos], key, mask=m)
            return ov + plsc.all_reduce_population_count(m)

        off_vec = lax.fori_loop(0, lax.div(n_ch + jnp.int32(15), jnp.int32(16)),
                                abody, off_vec)
    n1 = jnp.max(off_vec) + jnp.int32(1)

    def hist_pass(src, ntrips, nvalid, sh):
        nspl = splat(nvalid)

        def hbody(i, cc):
            key = src[pl.ds(i * 16, 16)]
            binv = lax.shift_right_logical(key ^ sign, jnp.int32(sh)) & jnp.int32(_NBINS - 1)
            m = (lane + i * 16) < nspl
            plsc.addupdate_scatter(hist, [lane * _NBINS + binv], ones16, mask=m)
            return cc

        lax.fori_loop(0, ntrips, hbody, jnp.int32(0))

    def zero_scatter(src, ntrips, nvalid, sh):
        nspl = splat(nvalid)

        def zbody(i, cc):
            key = src[pl.ds(i * 16, 16)]
            binv = lax.shift_right_logical(key ^ sign, jnp.int32(sh)) & jnp.int32(_NBINS - 1)
            m = (lane + i * 16) < nspl
            plsc.store_scatter(hist, [lane * _NBINS + binv], zeros16, mask=m)
            return cc

        lax.fori_loop(0, ntrips, zbody, jnp.int32(0))

    def scan_hist(k_rem):
        def sbody(i, csum):
            tot = hist[pl.ds(i * 16, 16)]
            for j in range(1, 16):
                tot = tot + hist[pl.ds(j * _NBINS + i * 16, 16)]
            cbuf[pl.ds(i * 16, 16)] = plsc.cumsum(tot) + csum
            return csum + jnp.sum(tot)

        n = lax.fori_loop(0, _NBINS // 16, sbody, jnp.int32(0))
        return level_select_from_cbuf(n, k_rem)

    def compact(src, dst, ntrips, nvalid, sh, bstar):
        nspl = splat(nvalid)

        def cbody(i, ov):
            key = src[pl.ds(i * 16, 16)]
            binv = lax.shift_right_logical(key ^ sign, jnp.int32(sh)) & jnp.int32(_NBINS - 1)
            m = (binv == bstar) & ((lane + i * 16) < nspl)
            mi = jnp.where(m, jnp.int32(1), jnp.int32(0))
            pos = jnp.maximum(plsc.cumsum(mi) + ov, jnp.int32(0))
            plsc.store_scatter(dst, [pos], key, mask=m)
            return ov + plsc.all_reduce_population_count(m)

        ov = lax.fori_loop(0, ntrips, cbody, jnp.full((16,), -1, jnp.int32))
        return jnp.max(ov) + jnp.int32(1)

    t1 = lax.div(n1 + jnp.int32(15), jnp.int32(16))
    zero_hist()
    hist_pass(ck1, t1, n1, 16)
    b1, g1 = scan_hist(k_rem)
    k_rem = k_rem - g1
    n2 = compact(ck1, ck2, t1, n1, 16, b1)
    t2 = lax.div(n2 + jnp.int32(15), jnp.int32(16))

    zero_scatter(ck1, t1, n1, 16)
    hist_pass(ck2, t2, n2, 8)
    b2, g2 = scan_hist(k_rem)
    k_rem = k_rem - g2
    n3 = compact(ck2, ck1, t2, n2, 8, b2)
    t3 = lax.div(n3 + jnp.int32(15), jnp.int32(16))

    zero_scatter(ck2, t2, n2, 8)
    hist_pass(ck1, t3, n3, 0)
    b3, g3 = scan_hist(k_rem)
    k_rem = k_rem - g3

    sl8 = jnp.int32(8)
    thresh = lax.shift_left(
        lax.shift_left(lax.shift_left(b0, sl8) | b1, sl8) | b2, sl8
    ) | b3
    thresh = thresh ^ sign
    r_ties = k_rem

    def cntbody(i, carry):
        cgt, ceq = carry
        for u in range(4):
            j = i * 4 + u
            key = keys[pl.ds(j * 16, 16)]
            cgt = cgt + plsc.all_reduce_population_count(key > thresh)
            ceq = ceq + plsc.all_reduce_population_count(key == thresh)
        return cgt, ceq

    cgt, ceq = lax.fori_loop(0, _NVC // 4, cntbody, (zeros16, zeros16))
    tmp[pl.ds(0, 16)] = cgt
    tmp[pl.ds(16, 16)] = ceq
    pltpu.sync_copy(tmp, sh_cnt.at[pl.ds(cid * 32, 32)])
    plsc.subcore_barrier()

    pltpu.sync_copy(sh_cnt.at[pl.ds(0, _NCH * 32)], cnt.at[pl.ds(0, _NCH * 32)])
    zero = jnp.int32(0)
    eq_pref = zero
    sel_pref = zero
    sel_prefs = []
    my_eq_pref = zero
    for ch in range(_NCH):
        sel_prefs.append(sel_pref)
        is_mine = cid == ch
        my_eq_pref = jnp.where(is_mine, eq_pref, my_eq_pref)
        g_ch = cnt[pl.ds(ch * 32, 16)][0]
        e_ch = cnt[pl.ds(ch * 32 + 16, 16)][0]
        tie_ch = jnp.minimum(jnp.maximum(r_ties - eq_pref, zero), e_ch)
        eq_pref = eq_pref + e_ch
        sel_pref = sel_pref + g_ch + tie_ch

    def fbody(i, carry):
        ceqv, cselv = carry
        for u in range(4):
            j = i * 4 + u
            key = keys[pl.ds(j * 16, 16)]
            gt = key > thresh
            eq = key == thresh
            eqi = jnp.where(eq, jnp.int32(1), jnp.int32(0))
            rank = plsc.cumsum(eqi) + ceqv
            sel = gt | (eq & (rank <= r_ties))
            seli = jnp.where(sel, jnp.int32(1), jnp.int32(0))
            mrow[pl.ds(j * 16, 16)] = seli
            pos = jnp.maximum(plsc.cumsum(seli) + cselv, jnp.int32(0))
            plsc.store_scatter(ibuf, [pos], lane + jnp.int32(j * 16) + cid * _C, mask=sel)
            ceqv = ceqv + plsc.all_reduce_population_count(eq)
            cselv = cselv + plsc.all_reduce_population_count(sel)
        return ceqv, cselv

    lax.fori_loop(0, _NVC // 4, fbody,
                  (jnp.full((16,), my_eq_pref, jnp.int32),
                   jnp.full((16,), -1, jnp.int32)))

    pltpu.sync_copy(mrow, mask_hbm.at[pl.ds(lr * _S + cid * _C, _C)])
    pltpu.sync_copy(ibuf, sh_idx.at[pl.ds(cid * _C, _C)])
    plsc.subcore_barrier()

    pltpu.sync_copy(sh_idx.at[pl.ds(0, _NCH * _C)], asm)
    for v in range(_SEGV):
        p = splat(cid * _SEG + v * 16) + lane
        sc = zeros16
        for ch in range(1, _NCH):
            sc = sc + jnp.where(p >= splat(sel_prefs[ch]), jnp.int32(1), jnp.int32(0))
        off = p
        for ch in range(_NCH):
            off = jnp.where(sc == ch, p - sel_prefs[ch] + jnp.int32(ch * _C), off)
        off = jnp.minimum(jnp.maximum(off, jnp.int32(0)), jnp.int32(_NCH * _C - 1))
        ibuf2[pl.ds(v * 16, 16)] = plsc.load_gather(asm, [off])

    pltpu.sync_copy(ibuf2.at[pl.ds(0, _SEG)],
                    idx_hbm.at[pl.ds(lr * _KPAD + cid * _SEG, _SEG)])


@functools.partial(
    pl.kernel,
    mesh=plsc.VectorSubcoreMesh(core_axis_name="c", subcore_axis_name="s"),
    compiler_params=pltpu.CompilerParams(needs_layout_passes=False),
    out_type=[
        jax.ShapeDtypeStruct((_HB * _S,), jnp.int32),
        jax.ShapeDtypeStruct((_HB * _KPAD,), jnp.int32),
    ],
    scratch_types=[
        pltpu.VMEM((_C,), jnp.float32),
        pltpu.VMEM((_C,), jnp.int32),
        pltpu.VMEM((16 * _NBINS,), jnp.int32),
        pltpu.VMEM((_NBINS,), jnp.int32),
        pltpu.VMEM((_C,), jnp.int32),
        pltpu.VMEM((_NCH * _C,), jnp.int32),
        pltpu.VMEM((_NCH * _C,), jnp.int32),
        pltpu.VMEM((_NCH * _C,), jnp.int32),
        pltpu.VMEM((_C,), jnp.int32),
        pltpu.VMEM((_C,), jnp.int32),
        pltpu.VMEM((_SEGV * 16,), jnp.int32),
        pltpu.VMEM((32,), jnp.int32),
        pltpu.VMEM((_NCH * 32,), jnp.int32),
        pltpu.VMEM_SHARED((_NCH * _NBINS,), jnp.int32),
        pltpu.VMEM_SHARED((_NCH * 16,), jnp.int32),
        pltpu.VMEM_SHARED((_NCH * _C,), jnp.int32),
        pltpu.VMEM_SHARED((_NCH * _C,), jnp.int32),
        pltpu.VMEM_SHARED((_NCH * 32,), jnp.int32),
    ],
)
def _select(logits_hbm, mask_hbm, idx_hbm,
            lrow, keys, hist, cbuf, lck, ck1, ck2, asm,
            mrow, ibuf, ibuf2, tmp, cnt,
            sh_hist, sh_nck, sh_ck, sh_idx, sh_cnt):
    _select_body(logits_hbm, mask_hbm, idx_hbm,
                 lrow, keys, hist, cbuf, lck, ck1, ck2, asm,
                 mrow, ibuf, ibuf2, tmp, cnt,
                 sh_hist, sh_nck, sh_ck, sh_idx, sh_cnt)


_gate_half0 = _make_gate(0)
_gate_half1 = _make_gate(1)


def kernel(x, gate_w, gate_b, log_temp):
    x2d = x.reshape(_B * _S, _D)
    w = gate_w.reshape(_D, 1)
    b = gate_b.reshape(1, 1)
    lgA, spA, entA = _gate_half0(x2d, w, b)
    lgB, spB, entB = _gate_half1(x2d, w, b)
    mA, iA = _select(lgA.reshape(_HB * _S))
    mB, iB = _select(lgB.reshape(_HB * _S))
    logits = jnp.concatenate([lgA, lgB], axis=0).reshape(_B, _S)
    mask = jnp.concatenate([mA, mB], axis=0).reshape(_B, _S).astype(jnp.bool_)
    indices = jnp.concatenate([iA, iB], axis=0).reshape(_B, _KPAD)[:, :_K]
    sp = spA[0, 0] + spB[0, 0]
    ent = entA[0, 0] + entB[0, 0]
    mean_p = sp / (_B * _S)
    aux = 0.1 * (mean_p - 0.2) ** 2 + 0.01 * (ent / (_B * _S))
    return mask, indices, logits, aux


# speedup vs baseline: 1.0754x; 1.0754x over previous
import functools

import jax
import jax.numpy as jnp
from jax import lax
from jax.experimental import pallas as pl
from jax.experimental.pallas import tpu as pltpu
from jax.experimental.pallas import tpu_sc as plsc

_B, _S, _D = 4, 8192, 768
_K = 1638
_KPAD = 1664
_SEG = _KPAD // 8
_BLK = 4096
_NBLK = (_B * _S) // _BLK
_C = 1024
_NVC = _C // 16
_NBINS = 256


def _gate_body(x_ref, w_ref, b_ref, out_ref, sp_ref, ent_ref):
    i = pl.program_id(0)
    lg = lax.dot_general(
        x_ref[...], w_ref[...], (((1,), (0,)), ((), ())),
        preferred_element_type=jnp.float32,
    ) + b_ref[0, 0]
    out_ref[...] = lg
    p = jax.nn.sigmoid(lg)
    ent = -(p * jnp.log(p + 1e-10) + (1.0 - p) * jnp.log(1.0 - p + 1e-10))
    sp = jnp.sum(p)
    en = jnp.sum(ent)

    @pl.when(i == 0)
    def _init():
        sp_ref[0, 0] = sp
        ent_ref[0, 0] = en

    @pl.when(i != 0)
    def _acc():
        sp_ref[0, 0] += sp
        ent_ref[0, 0] += en


def _gate(x2d, w, b):
    return pl.pallas_call(
        _gate_body,
        grid=(_NBLK,),
        in_specs=[
            pl.BlockSpec((_BLK, _D), lambda i: (i, 0)),
            pl.BlockSpec((_D, 1), lambda i: (0, 0)),
            pl.BlockSpec(memory_space=pltpu.SMEM),
        ],
        out_specs=[
            pl.BlockSpec((_BLK, 1), lambda i: (i, 0)),
            pl.BlockSpec(memory_space=pltpu.SMEM),
            pl.BlockSpec(memory_space=pltpu.SMEM),
        ],
        out_shape=[
            jax.ShapeDtypeStruct((_B * _S, 1), jnp.float32),
            jax.ShapeDtypeStruct((1, 1), jnp.float32),
            jax.ShapeDtypeStruct((1, 1), jnp.float32),
        ],
        compiler_params=pltpu.CompilerParams(
            dimension_semantics=("arbitrary",),
        ),
    )(x2d, w, b)


def _select_body(logits_hbm, mask_hbm, idx_hbm,
                 lrow, keys, hist, cbuf, lck, ck1, ck2, asm,
                 mrow, ibuf, ibuf2, tmp, cnt,
                 sh_hist, sh_nck, sh_ck, sh_idx, sh_cnt):
    s = lax.axis_index("s")
    c = lax.axis_index("c")
    rl = s // 8
    cid = s % 8
    row = 2 * c + rl
    slot = rl * 8 + cid

    lane = lax.iota(jnp.int32, 16)
    zeros16 = jnp.zeros((16,), jnp.int32)
    ones16 = jnp.ones((16,), jnp.int32)
    sign = jnp.int32(-2147483648)

    def zero_hist():
        def zbody(i, cc):
            hist[pl.ds(i * 16, 16)] = zeros16
            return cc
        lax.fori_loop(0, (16 * _NBINS) // 16, zbody, jnp.int32(0))

    def splat(x):
        return jnp.full((16,), x, jnp.int32)

    pltpu.sync_copy(logits_hbm.at[pl.ds(row * _S + cid * _C, _C)], lrow)
    zero_hist()

    def p1body(i, cc):
        for u in range(4):
            j = i * 4 + u
            bits = lax.bitcast_convert_type(lrow[pl.ds(j * 16, 16)], jnp.int32)
            key = bits ^ (lax.shift_right_arithmetic(bits, 31) & jnp.int32(0x7FFFFFFF))
            key = key + jnp.where(key == jnp.int32(-1), jnp.int32(1), jnp.int32(0))
            keys[pl.ds(j * 16, 16)] = key
            binv = lax.shift_right_logical(key ^ sign, jnp.int32(24))
            plsc.addupdate_scatter(hist, [lane * _NBINS + binv], ones16)
        return cc

    lax.fori_loop(0, _NVC // 4, p1body, jnp.int32(0))

    def rbody(i, cc):
        tot = hist[pl.ds(i * 16, 16)]
        for j in range(1, 16):
            tot = tot + hist[pl.ds(j * _NBINS + i * 16, 16)]
        cbuf[pl.ds(i * 16, 16)] = tot
        return cc

    lax.fori_loop(0, _NBINS // 16, rbody, jnp.int32(0))
    pltpu.sync_copy(cbuf, sh_hist.at[pl.ds(slot * _NBINS, _NBINS)])
    plsc.subcore_barrier()

    def level_select_from_cbuf(n, k_rem):
        excess = n - k_rem

        def bbody(i, acc):
            cc = cbuf[pl.ds(i * 16, 16)]
            return acc + jnp.where(cc <= excess, jnp.int32(1), jnp.int32(0))

        accv = lax.fori_loop(0, _NBINS // 16, bbody, zeros16)
        bstar = jnp.sum(accv)
        cb = plsc.load_gather(cbuf, [splat(bstar)])
        g_above = n - jnp.max(cb)
        return bstar, g_above

    pltpu.sync_copy(sh_hist.at[pl.ds(rl * 8 * _NBINS, 8 * _NBINS)],
                    asm.at[pl.ds(0, 8 * _NBINS)])

    def sbody0(i, csum):
        tot = asm[pl.ds(i * 16, 16)]
        for j in range(1, 8):
            tot = tot + asm[pl.ds(j * _NBINS + i * 16, 16)]
        cbuf[pl.ds(i * 16, 16)] = plsc.cumsum(tot) + csum
        return csum + jnp.sum(tot)

    n0 = lax.fori_loop(0, _NBINS // 16, sbody0, jnp.int32(0))
    b0, g0 = level_select_from_cbuf(n0, jnp.int32(_K))


    def c0body(i, off_vec):
        for u in range(4):
            j = i * 4 + u
            key = keys[pl.ds(j * 16, 16)]
            binv = lax.shift_right_logical(key ^ sign, jnp.int32(24))
            m = binv == b0
            mi = jnp.where(m, jnp.int32(1), jnp.int32(0))
            pos = jnp.maximum(plsc.cumsum(mi) + off_vec, jnp.int32(0))
            plsc.store_scatter(lck, [pos], key, mask=m)
            off_vec = off_vec + plsc.all_reduce_population_count(m)
        return off_vec

    offv = lax.fori_loop(0, _NVC // 4, c0body, jnp.full((16,), -1, jnp.int32))
    nck = jnp.max(offv) + jnp.int32(1)
    tmp[pl.ds(0, 16)] = splat(nck)
    pltpu.sync_copy(tmp.at[pl.ds(0, 16)], sh_nck.at[pl.ds(slot * 16, 16)])
    pltpu.sync_copy(lck, sh_ck.at[pl.ds(slot * _C, _C)])
    plsc.subcore_barrier()

    pltpu.sync_copy(sh_ck.at[pl.ds(rl * 8 * _C, 8 * _C)], asm)
    pltpu.sync_copy(sh_nck.at[pl.ds(rl * 8 * 16, 8 * 16)],
                    cnt.at[pl.ds(0, 8 * 16)])
    k_rem = jnp.int32(_K) - g0

    off_vec = jnp.full((16,), -1, jnp.int32)
    for ch in range(8):
        n_ch = cnt[pl.ds(ch * 16, 16)][0]
        nspl = splat(n_ch)

        def abody(i, ov, ch=ch, nspl=nspl):
            m = (lane + i * 16) < nspl
            key = asm[pl.ds(ch * _C + i * 16, 16)]
            mi = jnp.where(m, jnp.int32(1), jnp.int32(0))
            pos = jnp.maximum(plsc.cumsum(mi) + ov, jnp.int32(0))
            plsc.store_scatter(ck1, [pos], key, mask=m)
            return ov + plsc.all_reduce_population_count(m)

        off_vec = lax.fori_loop(0, lax.div(n_ch + jnp.int32(15), jnp.int32(16)),
                                abody, off_vec)
    n1 = jnp.max(off_vec) + jnp.int32(1)

    def hist_pass(src, ntrips, nvalid, sh):
        nspl = splat(nvalid)

        def hbody(i, cc):
            key = src[pl.ds(i * 16, 16)]
            binv = lax.shift_right_logical(key ^ sign, jnp.int32(sh)) & jnp.int32(_NBINS - 1)
            m = (lane + i * 16) < nspl
            plsc.addupdate_scatter(hist, [lane * _NBINS + binv], ones16, mask=m)
            return cc

        lax.fori_loop(0, ntrips, hbody, jnp.int32(0))

    def zero_scatter(src, ntrips, nvalid, sh):
        nspl = splat(nvalid)

        def zbody(i, cc):
            key = src[pl.ds(i * 16, 16)]
            binv = lax.shift_right_logical(key ^ sign, jnp.int32(sh)) & jnp.int32(_NBINS - 1)
            m = (lane + i * 16) < nspl
            plsc.store_scatter(hist, [lane * _NBINS + binv], zeros16, mask=m)
            return cc

        lax.fori_loop(0, ntrips, zbody, jnp.int32(0))

    def scan_hist(k_rem):
        def sbody(i, csum):
            tot = hist[pl.ds(i * 16, 16)]
            for j in range(1, 16):
                tot = tot + hist[pl.ds(j * _NBINS + i * 16, 16)]
            cbuf[pl.ds(i * 16, 16)] = plsc.cumsum(tot) + csum
            return csum + jnp.sum(tot)

        n = lax.fori_loop(0, _NBINS // 16, sbody, jnp.int32(0))
        return level_select_from_cbuf(n, k_rem)

    def compact(src, dst, ntrips, nvalid, sh, bstar):
        nspl = splat(nvalid)

        def cbody(i, ov):
            key = src[pl.ds(i * 16, 16)]
            binv = lax.shift_right_logical(key ^ sign, jnp.int32(sh)) & jnp.int32(_NBINS - 1)
            m = (binv == bstar) & ((lane + i * 16) < nspl)
            mi = jnp.where(m, jnp.int32(1), jnp.int32(0))
            pos = jnp.maximum(plsc.cumsum(mi) + ov, jnp.int32(0))
            plsc.store_scatter(dst, [pos], key, mask=m)
            return ov + plsc.all_reduce_population_count(m)

        ov = lax.fori_loop(0, ntrips, cbody, jnp.full((16,), -1, jnp.int32))
        return jnp.max(ov) + jnp.int32(1)

    t1 = lax.div(n1 + jnp.int32(15), jnp.int32(16))
    zero_hist()
    hist_pass(ck1, t1, n1, 16)
    b1, g1 = scan_hist(k_rem)
    k_rem = k_rem - g1
    n2 = compact(ck1, ck2, t1, n1, 16, b1)
    t2 = lax.div(n2 + jnp.int32(15), jnp.int32(16))

    zero_scatter(ck1, t1, n1, 16)
    hist_pass(ck2, t2, n2, 8)
    b2, g2 = scan_hist(k_rem)
    k_rem = k_rem - g2
    n3 = compact(ck2, ck1, t2, n2, 8, b2)
    t3 = lax.div(n3 + jnp.int32(15), jnp.int32(16))

    zero_scatter(ck2, t2, n2, 8)
    hist_pass(ck1, t3, n3, 0)
    b3, g3 = scan_hist(k_rem)
    k_rem = k_rem - g3

    sl8 = jnp.int32(8)
    thresh = lax.shift_left(
        lax.shift_left(lax.shift_left(b0, sl8) | b1, sl8) | b2, sl8
    ) | b3
    thresh = thresh ^ sign
    r_ties = k_rem


    def cntbody(i, carry):
        cgt, ceq = carry
        for u in range(4):
            j = i * 4 + u
            key = keys[pl.ds(j * 16, 16)]
            cgt = cgt + plsc.all_reduce_population_count(key > thresh)
            ceq = ceq + plsc.all_reduce_population_count(key == thresh)
        return cgt, ceq

    cgt, ceq = lax.fori_loop(0, _NVC // 4, cntbody, (zeros16, zeros16))
    tmp[pl.ds(0, 16)] = cgt
    tmp[pl.ds(16, 16)] = ceq
    pltpu.sync_copy(tmp, sh_cnt.at[pl.ds(slot * 32, 32)])
    plsc.subcore_barrier()

    pltpu.sync_copy(sh_cnt.at[pl.ds(rl * 8 * 32, 8 * 32)], cnt)
    zero = jnp.int32(0)
    eq_pref = zero
    sel_pref = zero
    sel_prefs = []
    my_eq_pref = zero
    my_sel_pref = zero
    for ch in range(8):
        sel_prefs.append(sel_pref)
        is_mine = cid == ch
        my_eq_pref = jnp.where(is_mine, eq_pref, my_eq_pref)
        my_sel_pref = jnp.where(is_mine, sel_pref, my_sel_pref)
        g_ch = cnt[pl.ds(ch * 32, 16)][0]
        e_ch = cnt[pl.ds(ch * 32 + 16, 16)][0]
        tie_ch = jnp.minimum(jnp.maximum(r_ties - eq_pref, zero), e_ch)
        eq_pref = eq_pref + e_ch
        sel_pref = sel_pref + g_ch + tie_ch

    def fbody(i, carry):
        ceqv, cselv = carry
        for u in range(4):
            j = i * 4 + u
            key = keys[pl.ds(j * 16, 16)]
            gt = key > thresh
            eq = key == thresh
            eqi = jnp.where(eq, jnp.int32(1), jnp.int32(0))
            rank = plsc.cumsum(eqi) + ceqv
            sel = gt | (eq & (rank <= r_ties))
            seli = jnp.where(sel, jnp.int32(1), jnp.int32(0))
            mrow[pl.ds(j * 16, 16)] = seli
            pos = jnp.maximum(plsc.cumsum(seli) + cselv, jnp.int32(0))
            plsc.store_scatter(ibuf, [pos], lane + jnp.int32(j * 16) + cid * _C, mask=sel)
            ceqv = ceqv + plsc.all_reduce_population_count(eq)
            cselv = cselv + plsc.all_reduce_population_count(sel)
        return ceqv, cselv

    lax.fori_loop(0, _NVC // 4, fbody,
                  (jnp.full((16,), my_eq_pref, jnp.int32),
                   jnp.full((16,), -1, jnp.int32)))

    pltpu.sync_copy(mrow, mask_hbm.at[pl.ds(row * _S + cid * _C, _C)])
    pltpu.sync_copy(ibuf, sh_idx.at[pl.ds(slot * _C, _C)])
    plsc.subcore_barrier()

    pltpu.sync_copy(sh_idx.at[pl.ds(rl * 8 * _C, 8 * _C)], asm)
    for v in range(_SEG // 16):
        p = splat(cid * _SEG + v * 16) + lane
        sc = zeros16
        for ch in range(1, 8):
            sc = sc + jnp.where(p >= splat(sel_prefs[ch]), jnp.int32(1), jnp.int32(0))
        off = p
        for ch in range(8):
            off = jnp.where(sc == ch, p - sel_prefs[ch] + jnp.int32(ch * _C), off)
        off = jnp.minimum(jnp.maximum(off, jnp.int32(0)), jnp.int32(8 * _C - 1))
        ibuf2[pl.ds(v * 16, 16)] = plsc.load_gather(asm, [off])

    pltpu.sync_copy(ibuf2, idx_hbm.at[pl.ds(row * _KPAD + cid * _SEG, _SEG)])


@functools.partial(
    pl.kernel,
    mesh=plsc.VectorSubcoreMesh(core_axis_name="c", subcore_axis_name="s"),
    compiler_params=pltpu.CompilerParams(needs_layout_passes=False),
    out_type=[
        jax.ShapeDtypeStruct((_B * _S,), jnp.int32),
        jax.ShapeDtypeStruct((_B * _KPAD,), jnp.int32),
    ],
    scratch_types=[
        pltpu.VMEM((_C,), jnp.float32),
        pltpu.VMEM((_C,), jnp.int32),
        pltpu.VMEM((16 * _NBINS,), jnp.int32),
        pltpu.VMEM((_NBINS,), jnp.int32),
        pltpu.VMEM((_C,), jnp.int32),
        pltpu.VMEM((8 * _C,), jnp.int32),
        pltpu.VMEM((8 * _C,), jnp.int32),
        pltpu.VMEM((8 * _C,), jnp.int32),
        pltpu.VMEM((_C,), jnp.int32),
        pltpu.VMEM((_C,), jnp.int32),
        pltpu.VMEM((_SEG,), jnp.int32),
        pltpu.VMEM((32,), jnp.int32),
        pltpu.VMEM((8 * 32,), jnp.int32),
        pltpu.VMEM_SHARED((2 * 8 * _NBINS,), jnp.int32),
        pltpu.VMEM_SHARED((2 * 8 * 16,), jnp.int32),
        pltpu.VMEM_SHARED((2 * 8 * _C,), jnp.int32),
        pltpu.VMEM_SHARED((2 * 8 * _C,), jnp.int32),
        pltpu.VMEM_SHARED((2 * 8 * 32,), jnp.int32),
    ],
)
def _select(logits_hbm, mask_hbm, idx_hbm,
            lrow, keys, hist, cbuf, lck, ck1, ck2, asm,
            mrow, ibuf, ibuf2, tmp, cnt,
            sh_hist, sh_nck, sh_ck, sh_idx, sh_cnt):
    _select_body(logits_hbm, mask_hbm, idx_hbm,
                 lrow, keys, hist, cbuf, lck, ck1, ck2, asm,
                 mrow, ibuf, ibuf2, tmp, cnt,
                 sh_hist, sh_nck, sh_ck, sh_idx, sh_cnt)


def kernel(x, gate_w, gate_b, log_temp):
    x2d = x.reshape(_B * _S, _D)
    w = gate_w.reshape(_D, 1)
    b = gate_b.reshape(1, 1)
    logits2d, sp, ent = _gate(x2d, w, b)
    logits = logits2d.reshape(_B, _S)
    mask_i, idx_p = _select(logits2d.reshape(_B * _S))
    mask = mask_i.reshape(_B, _S).astype(jnp.bool_)
    indices = idx_p.reshape(_B, _KPAD)[:, :_K]
    mean_p = sp[0, 0] / (_B * _S)
    aux = 0.1 * (mean_p - 0.2) ** 2 + 0.01 * (ent[0, 0] / (_B * _S))
    return mask, indices, logits, aux
